# Initial kernel scaffold; baseline (speedup 1.0000x reference)
import jax
import jax.numpy as jnp
from jax.experimental import pallas as pl


def kernel(x, edge_index, batch, W_root1, W_nbr1, W_root2, W_nbr2, W_out):
    raise NotImplementedError("write your pallas kernel here")


# speedup vs baseline: 7.0812x; 7.0812x over previous
import functools

import jax
import jax.numpy as jnp
from jax import lax
from jax.experimental import pallas as pl
from jax.experimental.pallas import tpu as pltpu
from jax.experimental.pallas import tpu_sc as plsc

N = 10000
E = 320000
D = 128
G = 64

NC = 2
NS = 16
NW = NC * NS
T = E // NW
K = 80
NCH = T // K
NZ = 5
RPZ = N // NZ

_sc_mesh = plsc.VectorSubcoreMesh(core_axis_name="c", subcore_axis_name="s")


@functools.partial(
    pl.kernel,
    mesh=_sc_mesh,
    out_type=jax.ShapeDtypeStruct((NC, N, D), jnp.float32),
    scratch_types=[
        pltpu.VMEM((NCH, K), jnp.int32),
        pltpu.VMEM((NCH, K), jnp.int32),
        pltpu.VMEM((K, D), jnp.float32),
        pltpu.VMEM_SHARED((N, D), jnp.float32),
        pltpu.SemaphoreType.DMA,
    ],
)
def _sc_scatter(g_hbm, src_hbm, dst_hbm, out_hbm, src_v, dst_v, buf, acc, sem):
    c = lax.axis_index("c")
    s = lax.axis_index("s")
    wid = c * NS + s

    pltpu.sync_copy(src_hbm.at[wid], src_v)
    pltpu.sync_copy(dst_hbm.at[wid], dst_v)

    zero = jnp.zeros((16,), jnp.float32)

    @pl.when(s < NZ)
    def _zero():
        def zrow(i, carry):
            for kk in range(D // 16):
                buf[i, pl.ds(kk * 16, 16)] = zero
            return carry

        lax.fori_loop(0, K, zrow, 0)
        for r in range(RPZ // K):
            pltpu.sync_copy(buf, acc.at[pl.ds(s * RPZ + r * K, K)])

    plsc.subcore_barrier()

    def chunk(j, carry):
        pltpu.async_copy(g_hbm.at[src_v.at[j]], buf, sem).wait()
        pltpu.sync_copy(buf, acc.at[dst_v.at[j]], add=True)
        return carry

    lax.fori_loop(0, NCH, chunk, 0)
    plsc.subcore_barrier()

    @pl.when(s < NZ)
    def _readout():
        pltpu.sync_copy(acc.at[pl.ds(s * RPZ, RPZ)], out_hbm.at[c, pl.ds(s * RPZ, RPZ)])


BR = 1000


def _mm2_body(x_ref, wr_ref, wn_ref, r_ref, g_ref):
    xb = x_ref[...]
    r_ref[...] = jnp.dot(xb, wr_ref[...], preferred_element_type=jnp.float32)
    g_ref[...] = jnp.dot(xb, wn_ref[...], preferred_element_type=jnp.float32)


def _fuse_mm2_body(r_ref, a0_ref, a1_ref, wr_ref, wn_ref, r2_ref, g2_ref):
    h = jnp.maximum(r_ref[...] + a0_ref[...] + a1_ref[...], 0.0)
    r2_ref[...] = jnp.dot(h, wr_ref[...], preferred_element_type=jnp.float32)
    g2_ref[...] = jnp.dot(h, wn_ref[...], preferred_element_type=jnp.float32)


def _pool_out_body(r2_ref, a0_ref, a1_ref, b_ref, wout_ref, o_ref):
    h = jnp.maximum(r2_ref[...] + a0_ref[...] + a1_ref[...], 0.0)
    lbl = lax.broadcasted_iota(jnp.int32, (G, N), 0).astype(jnp.float32)
    oh = (lbl == b_ref[...]).astype(jnp.float32)
    sums = jnp.dot(oh, h, preferred_element_type=jnp.float32)
    cnt = jnp.sum(oh, axis=1, keepdims=True)
    pooled = sums / jnp.maximum(cnt, 1.0)
    o_ref[...] = jnp.dot(pooled, wout_ref[...], preferred_element_type=jnp.float32)


_row_spec = pl.BlockSpec((BR, D), lambda i: (i, 0))
_w_spec = pl.BlockSpec((D, D), lambda i: (0, 0))
_nd_shape = jax.ShapeDtypeStruct((N, D), jnp.float32)

_mm2 = pl.pallas_call(
    _mm2_body,
    grid=(N // BR,),
    in_specs=[_row_spec, _w_spec, _w_spec],
    out_specs=[_row_spec, _row_spec],
    out_shape=[_nd_shape, _nd_shape],
)

_fuse_mm2 = pl.pallas_call(
    _fuse_mm2_body,
    grid=(N // BR,),
    in_specs=[_row_spec, _row_spec, _row_spec, _w_spec, _w_spec],
    out_specs=[_row_spec, _row_spec],
    out_shape=[_nd_shape, _nd_shape],
)

_pool_out = pl.pallas_call(
    _pool_out_body,
    out_shape=jax.ShapeDtypeStruct((G, D), jnp.float32),
)


def kernel(x, edge_index, batch, W_root1, W_nbr1, W_root2, W_nbr2, W_out):
    src = edge_index[0].astype(jnp.int32).reshape(NW, NCH, K)
    dst = edge_index[1].astype(jnp.int32).reshape(NW, NCH, K)
    b_row = batch.astype(jnp.float32).reshape(1, N)

    r1, g1 = _mm2(x, W_root1, W_nbr1)
    agg1 = _sc_scatter(g1, src, dst)
    r2, g2 = _fuse_mm2(r1, agg1[0], agg1[1], W_root2, W_nbr2)
    agg2 = _sc_scatter(g2, src, dst)
    return _pool_out(r2, agg2[0], agg2[1], b_row, W_out)
